# Initial kernel scaffold; baseline (speedup 1.0000x reference)
import jax
import jax.numpy as jnp
from jax.experimental import pallas as pl


def kernel(x, edge_index, edge_type, weights):
    raise NotImplementedError("write your pallas kernel here")


# speedup vs baseline: 2.7134x; 2.7134x over previous
import functools

import jax
import jax.numpy as jnp
from jax import lax
from jax.experimental import pallas as pl
from jax.experimental.pallas import tpu as pltpu
from jax.experimental.pallas import tpu_sc as plsc

_D = 128
_LANES = 16
_BLOCK_E = 80


def _normalize_body(x_ref, o_ref):
    xv = x_ref[...]
    ss = jnp.sum(xv * xv, axis=1, keepdims=True)
    o_ref[...] = xv * lax.rsqrt(ss)


def _normalize(x):
    return pl.pallas_call(
        _normalize_body,
        out_shape=jax.ShapeDtypeStruct(x.shape, x.dtype),
    )(x)


@functools.lru_cache(maxsize=None)
def _make_sc_kernel(n_edges, num_cores, num_subcores):
    n_workers = num_cores * num_subcores
    e_per_w = n_edges // n_workers
    nb = e_per_w // _BLOCK_E
    mesh = plsc.VectorSubcoreMesh(core_axis_name="c", subcore_axis_name="s")

    @functools.partial(
        pl.kernel,
        mesh=mesh,
        out_type=jax.ShapeDtypeStruct((n_edges,), jnp.float32),
        scratch_types=[
            pltpu.VMEM((_BLOCK_E,), jnp.int32),
            pltpu.VMEM((_BLOCK_E,), jnp.int32),
            pltpu.VMEM((_BLOCK_E,), jnp.int32),
            pltpu.VMEM((_BLOCK_E, _D), jnp.float32),
            pltpu.VMEM((_BLOCK_E, _D), jnp.float32),
            pltpu.VMEM((_BLOCK_E, _D), jnp.float32),
            pltpu.VMEM((_BLOCK_E,), jnp.float32),
            pltpu.SemaphoreType.DMA,
            pltpu.SemaphoreType.DMA,
            pltpu.SemaphoreType.DMA,
        ],
    )
    def k(xn_hbm, w_hbm, src_hbm, dst_hbm, rel_hbm, out_hbm,
          si_v, di_v, ri_v, s_v, o_v, r_v, out_v, sem_s, sem_o, sem_r):
        wid = lax.axis_index("s") * num_cores + lax.axis_index("c")
        lanes = lax.iota(jnp.int32, _LANES)

        perm_dnums = lax.GatherDimensionNumbers(
            offset_dims=(), collapsed_slice_dims=(0,), start_index_map=(0,))

        def lane_sum(v):
            for sh in (8, 4, 2, 1):
                p = lax.gather(v, (lanes ^ sh)[:, None], perm_dnums,
                               slice_sizes=(1,),
                               mode=lax.GatherScatterMode.PROMISE_IN_BOUNDS)
                v = v + p
            return v

        def block_body(b, carry):
            base = wid * e_per_w + b * _BLOCK_E
            pltpu.sync_copy(src_hbm.at[pl.ds(base, _BLOCK_E)], si_v)
            pltpu.sync_copy(dst_hbm.at[pl.ds(base, _BLOCK_E)], di_v)
            pltpu.sync_copy(rel_hbm.at[pl.ds(base, _BLOCK_E)], ri_v)
            cs = pltpu.async_copy(xn_hbm.at[si_v], s_v, sem_s)
            co = pltpu.async_copy(xn_hbm.at[di_v], o_v, sem_o)
            cr = pltpu.async_copy(w_hbm.at[ri_v], r_v, sem_r)
            cs.wait()
            co.wait()
            cr.wait()

            def grp_body(g, c2):
                scores = jnp.zeros((_LANES,), jnp.float32)
                for j in range(_LANES):
                    e = g * _LANES + j
                    acc = jnp.zeros((_LANES,), jnp.float32)
                    for kk in range(_D // _LANES):
                        sl = pl.ds(kk * _LANES, _LANES)
                        acc = acc + jnp.abs(s_v[e, sl] + r_v[e, sl] - o_v[e, sl])
                    tot = lane_sum(acc)
                    scores = jnp.where(lanes == j, -tot, scores)
                out_v[pl.ds(g * _LANES, _LANES)] = scores
                return c2

            lax.fori_loop(0, _BLOCK_E // _LANES, grp_body, 0)
            pltpu.sync_copy(out_v, out_hbm.at[pl.ds(base, _BLOCK_E)])
            return carry

        lax.fori_loop(0, nb, block_body, 0)

    return k


def kernel(x, edge_index, edge_type, weights):
    x = x.astype(jnp.float32)
    weights = weights.astype(jnp.float32)
    src = edge_index[0].astype(jnp.int32)
    dst = edge_index[1].astype(jnp.int32)
    rel = edge_type.astype(jnp.int32)
    xn = _normalize(x)
    info = plsc.get_sparse_core_info()
    k = _make_sc_kernel(src.shape[0], info.num_cores, info.num_subcores)
    return k(xn, weights, src, dst, rel)


# speedup vs baseline: 4.0052x; 1.4761x over previous
import functools

import jax
import jax.numpy as jnp
from jax import lax
from jax.experimental import pallas as pl
from jax.experimental.pallas import tpu as pltpu
from jax.experimental.pallas import tpu_sc as plsc

_D = 128
_LANES = 16
_BLOCK_E = 80


def _normalize_body(x_ref, o_ref):
    xv = x_ref[...]
    ss = jnp.sum(xv * xv, axis=1, keepdims=True)
    o_ref[...] = xv * lax.rsqrt(ss)


def _normalize(x):
    return pl.pallas_call(
        _normalize_body,
        out_shape=jax.ShapeDtypeStruct(x.shape, x.dtype),
    )(x)


@functools.lru_cache(maxsize=None)
def _make_sc_kernel(n_edges, num_cores, num_subcores):
    n_workers = num_cores * num_subcores
    e_per_w = n_edges // n_workers
    nb = e_per_w // _BLOCK_E
    mesh = plsc.VectorSubcoreMesh(core_axis_name="c", subcore_axis_name="s")

    @functools.partial(
        pl.kernel,
        mesh=mesh,
        out_type=jax.ShapeDtypeStruct((n_edges,), jnp.float32),
        scratch_types=[
            pltpu.VMEM((e_per_w,), jnp.int32),
            pltpu.VMEM((e_per_w,), jnp.int32),
            pltpu.VMEM((e_per_w,), jnp.int32),
            pltpu.VMEM((2, _BLOCK_E, _D), jnp.float32),
            pltpu.VMEM((2, _BLOCK_E, _D), jnp.float32),
            pltpu.VMEM((2, _BLOCK_E, _D), jnp.float32),
            pltpu.VMEM((_BLOCK_E,), jnp.float32),
            pltpu.SemaphoreType.DMA,
            pltpu.SemaphoreType.DMA,
        ],
    )
    def k(xn_hbm, w_hbm, src_hbm, dst_hbm, rel_hbm, out_hbm,
          si_v, di_v, ri_v, s_v, o_v, r_v, out_v, sem0, sem1):
        wid = lax.axis_index("s") * num_cores + lax.axis_index("c")
        base_w = wid * e_per_w
        lanes = lax.iota(jnp.int32, _LANES)

        perm_dnums = lax.GatherDimensionNumbers(
            offset_dims=(), collapsed_slice_dims=(0,), start_index_map=(0,))

        def bf(v, sh):
            p = lax.gather(v, (lanes ^ sh)[:, None], perm_dnums,
                           slice_sizes=(1,),
                           mode=lax.GatherScatterMode.PROMISE_IN_BOUNDS)
            return v + p

        masks = {sh: (lanes & sh) == 0 for sh in (8, 4, 2, 1)}

        def issue(b, p):
            ib = pl.ds(b * _BLOCK_E, _BLOCK_E)
            sem = sem0 if p == 0 else sem1
            pltpu.async_copy(xn_hbm.at[si_v.at[ib]], s_v.at[p], sem)
            pltpu.async_copy(xn_hbm.at[di_v.at[ib]], o_v.at[p], sem)
            pltpu.async_copy(w_hbm.at[ri_v.at[ib]], r_v.at[p], sem)

        def drain(p):
            ib = pl.ds(0, _BLOCK_E)
            sem = sem0 if p == 0 else sem1
            pltpu.make_async_copy(xn_hbm.at[si_v.at[ib]], s_v.at[p], sem).wait()
            pltpu.make_async_copy(xn_hbm.at[di_v.at[ib]], o_v.at[p], sem).wait()
            pltpu.make_async_copy(w_hbm.at[ri_v.at[ib]], r_v.at[p], sem).wait()

        def compute(b, p):
            sb = s_v.at[p]
            ob = o_v.at[p]
            rb = r_v.at[p]

            def grp_body(g, c2):
                vecs = []
                for j in range(_LANES):
                    e = g * _LANES + j
                    acc = jnp.zeros((_LANES,), jnp.float32)
                    for kk in range(_D // _LANES):
                        sl = pl.ds(kk * _LANES, _LANES)
                        acc = acc + jnp.abs(sb[e, sl] + rb[e, sl] - ob[e, sl])
                    vecs.append(acc)
                for sh in (8, 4, 2, 1):
                    n = len(vecs) // 2
                    vecs = [jnp.where(masks[sh], bf(vecs[j], sh),
                                      bf(vecs[j + n], sh))
                            for j in range(n)]
                out_v[pl.ds(g * _LANES, _LANES)] = -vecs[0]
                return c2

            lax.fori_loop(0, _BLOCK_E // _LANES, grp_body, 0)
            pltpu.sync_copy(
                out_v, out_hbm.at[pl.ds(base_w + b * _BLOCK_E, _BLOCK_E)])

        iw = pl.ds(base_w, e_per_w)
        pltpu.sync_copy(src_hbm.at[iw], si_v)
        pltpu.sync_copy(dst_hbm.at[iw], di_v)
        pltpu.sync_copy(rel_hbm.at[iw], ri_v)

        issue(0, 0)
        issue(1, 1)

        def pair_body(gp, c):
            for j in range(2):
                b = gp * 2 + j
                drain(j)
                compute(b, j)
                nxt = b + 2

                @pl.when(nxt < nb)
                def _():
                    issue(nxt, j)
            return c

        lax.fori_loop(0, nb // 2, pair_body, 0)
        if nb % 2 == 1:
            drain(0)
            compute(nb - 1, 0)

    return k


def kernel(x, edge_index, edge_type, weights):
    x = x.astype(jnp.float32)
    weights = weights.astype(jnp.float32)
    src = edge_index[0].astype(jnp.int32)
    dst = edge_index[1].astype(jnp.int32)
    rel = edge_type.astype(jnp.int32)
    xn = _normalize(x)
    info = plsc.get_sparse_core_info()
    k = _make_sc_kernel(src.shape[0], info.num_cores, info.num_subcores)
    return k(xn, weights, src, dst, rel)


# speedup vs baseline: 4.5846x; 1.1447x over previous
import functools

import jax
import jax.numpy as jnp
from jax import lax
from jax.experimental import pallas as pl
from jax.experimental.pallas import tpu as pltpu
from jax.experimental.pallas import tpu_sc as plsc

_D = 128
_LANES = 16
_BLOCK_E = 16

_PERM_DNUMS = lax.GatherDimensionNumbers(
    offset_dims=(), collapsed_slice_dims=(0,), start_index_map=(0,)
)


def _perm(v, idx):
    return lax.gather(
        v,
        idx[:, None],
        _PERM_DNUMS,
        (1,),
        mode=lax.GatherScatterMode.PROMISE_IN_BOUNDS,
    )


def _normalize_body(x_ref, o_ref):
    xv = x_ref[...]
    ss = jnp.sum(xv * xv, axis=1, keepdims=True)
    o_ref[...] = xv * lax.rsqrt(ss)


def _normalize(x):
    return pl.pallas_call(
        _normalize_body,
        out_shape=jax.ShapeDtypeStruct(x.shape, jnp.float32),
    )(x)


@functools.lru_cache(maxsize=None)
def _make_sc_kernel(n_edges, num_cores, num_subcores):
    n_workers = num_cores * num_subcores
    e_per_w = n_edges // n_workers
    nb = e_per_w // _BLOCK_E
    mesh = plsc.VectorSubcoreMesh(core_axis_name="c", subcore_axis_name="s")

    @functools.partial(
        pl.kernel,
        mesh=mesh,
        out_type=jax.ShapeDtypeStruct((n_edges,), jnp.float32),
        scratch_types=[
            pltpu.VMEM((e_per_w,), jnp.int32),
            pltpu.VMEM((e_per_w,), jnp.int32),
            pltpu.VMEM((e_per_w,), jnp.int32),
            pltpu.VMEM((2, _BLOCK_E, _D), jnp.float32),
            pltpu.VMEM((2, _BLOCK_E, _D), jnp.float32),
            pltpu.VMEM((2, _BLOCK_E, _D), jnp.float32),
            pltpu.VMEM((e_per_w,), jnp.float32),
            pltpu.SemaphoreType.DMA,
            pltpu.SemaphoreType.DMA,
        ],
    )
    def k(xn_hbm, w_hbm, src_hbm, dst_hbm, rel_hbm, out_hbm,
          si_v, di_v, ri_v, s_v, o_v, r_v, acc_v, sem0, sem1):
        wid = lax.axis_index("s") * num_cores + lax.axis_index("c")
        base_w = wid * e_per_w
        lane = lax.iota(jnp.int32, _LANES)

        def issue(b, p):
            ib = pl.ds(b * _BLOCK_E, _BLOCK_E)
            sem = sem0 if p == 0 else sem1
            pltpu.async_copy(xn_hbm.at[si_v.at[ib]], s_v.at[p], sem)
            pltpu.async_copy(xn_hbm.at[di_v.at[ib]], o_v.at[p], sem)
            pltpu.async_copy(w_hbm.at[ri_v.at[ib]], r_v.at[p], sem)

        def drain(p):
            ib = pl.ds(0, _BLOCK_E)
            sem = sem0 if p == 0 else sem1
            pltpu.make_async_copy(xn_hbm.at[si_v.at[ib]], s_v.at[p], sem).wait()
            pltpu.make_async_copy(xn_hbm.at[di_v.at[ib]], o_v.at[p], sem).wait()
            pltpu.make_async_copy(w_hbm.at[ri_v.at[ib]], r_v.at[p], sem).wait()

        def compute(b, p):
            vecs = []
            for e in range(_BLOCK_E):
                acc = None
                for c in range(_D // _LANES):
                    sl = pl.ds(c * _LANES, _LANES)
                    d = jnp.abs(s_v[p, e, sl] + r_v[p, e, sl] - o_v[p, e, sl])
                    acc = d if acc is None else acc + d
                vecs.append(acc)
            for kbit in range(4):
                step = 1 << kbit
                mask = (lane & step) == 0
                pidx = lane ^ step
                nxt = []
                for i in range(0, len(vecs), 2):
                    a, bvec = vecs[i], vecs[i + 1]
                    u = jnp.where(mask, a, bvec)
                    w = jnp.where(mask, bvec, a)
                    nxt.append(u + _perm(w, pidx))
                vecs = nxt
            acc_v[pl.ds(b * _BLOCK_E, _BLOCK_E)] = -vecs[0]

        iw = pl.ds(base_w, e_per_w)
        pltpu.sync_copy(src_hbm.at[iw], si_v)
        pltpu.sync_copy(dst_hbm.at[iw], di_v)
        pltpu.sync_copy(rel_hbm.at[iw], ri_v)

        issue(0, 0)
        if nb % 2 == 1:
            m = (nb - 1) // 2

            def pair_body(i, c):
                b = 2 * i
                issue(b + 1, 1)
                drain(0)
                compute(b, 0)
                issue(b + 2, 0)
                drain(1)
                compute(b + 1, 1)
                return c

            if m > 0:
                lax.fori_loop(0, m, pair_body, 0)
            drain(0)
            compute(nb - 1, 0)
        else:
            m = nb // 2 - 1

            def pair_body(i, c):
                b = 2 * i
                issue(b + 1, 1)
                drain(0)
                compute(b, 0)
                issue(b + 2, 0)
                drain(1)
                compute(b + 1, 1)
                return c

            if m > 0:
                lax.fori_loop(0, m, pair_body, 0)
            issue(nb - 1, 1)
            drain(0)
            compute(nb - 2, 0)
            drain(1)
            compute(nb - 1, 1)

        pltpu.sync_copy(acc_v, out_hbm.at[pl.ds(base_w, e_per_w)])

    return k


def kernel(x, edge_index, edge_type, weights):
    x = x.astype(jnp.float32)
    w = weights.astype(jnp.float32)
    src = edge_index[0].astype(jnp.int32)
    dst = edge_index[1].astype(jnp.int32)
    rel = edge_type.astype(jnp.int32)
    xn = _normalize(x)
    info = plsc.get_sparse_core_info()
    k = _make_sc_kernel(src.shape[0], info.num_cores, info.num_subcores)
    return k(xn, w, src, dst, rel)
